# Initial kernel scaffold; baseline (speedup 1.0000x reference)
import jax
import jax.numpy as jnp
from jax.experimental import pallas as pl


def kernel(input_ids, segment_ids, token_table, seg_table, pe):
    raise NotImplementedError("write your pallas kernel here")


# speedup vs baseline: 1.8906x; 1.8906x over previous
import functools
import jax
import jax.numpy as jnp
from jax import lax
from jax.experimental import pallas as pl
from jax.experimental.pallas import tpu as pltpu
from jax.experimental.pallas import tpu_sc as plsc

D = 64
L = 200
NC = 2
NS = 16
NW = NC * NS

CT = 512
CG = CT // 128


def _body(ids_hbm, seg_hbm, tok_hbm, ps_hbm, out_hbm,
          idx_v, segv_v, idx2_v, tok_v, ps_v, sem0, sem1):
    wid = lax.axis_index("s") * NC + lax.axis_index("c")
    n_tok = ids_hbm.shape[0] * 128
    nt = n_tok // NW
    n_chunks = nt // CT
    w_row0 = wid * (nt // 128)

    iota16 = lax.iota(jnp.int32, 16)

    def chunk(c, carry):
        row0 = w_row0 + c * CG
        pltpu.sync_copy(ids_hbm.at[pl.ds(row0, CG)], idx_v)
        pltpu.sync_copy(seg_hbm.at[pl.ds(row0, CG)], segv_v)

        for g in range(CG):
            for u in range(8):
                fb = (row0 + g) * 128 + u * 16
                lv = lax.rem(iota16 + fb, L)
                sl = pl.ds(u * 16, 16)
                idx2_v[g, sl] = segv_v[g, sl] * L + lv

        cps = []
        for g in range(CG):
            cps.append(pltpu.async_copy(
                tok_hbm.at[idx_v.at[g]], tok_v.at[pl.ds(g * 128, 128)], sem0))
            cps.append(pltpu.async_copy(
                ps_hbm.at[idx2_v.at[g]], ps_v.at[pl.ds(g * 128, 128)], sem1))
        for cp in cps:
            cp.wait()

        def add_row(t, carry2):
            for k in range(4):
                sl = pl.ds(k * 16, 16)
                tok_v[t, sl] = tok_v[t, sl] + ps_v[t, sl]
            return carry2
        lax.fori_loop(0, CT, add_row, 0, unroll=2)

        pltpu.sync_copy(tok_v, out_hbm.at[pl.ds(row0 * 128, CT)])
        return carry

    lax.fori_loop(0, n_chunks, chunk, 0)


@jax.jit
def _sc_call(ids2d, seg2d, token_table, posseg):
    n_tok = ids2d.shape[0] * 128
    mesh = plsc.VectorSubcoreMesh(core_axis_name="c", subcore_axis_name="s")
    f = pl.kernel(
        _body,
        out_type=jax.ShapeDtypeStruct((n_tok, D), jnp.float32),
        mesh=mesh,
        compiler_params=pltpu.CompilerParams(use_tc_tiling_on_sc=False),
        scratch_types=[
            pltpu.VMEM((CG, 128), jnp.int32),
            pltpu.VMEM((CG, 128), jnp.int32),
            pltpu.VMEM((CG, 128), jnp.int32),
            pltpu.VMEM((CT, D), jnp.float32),
            pltpu.VMEM((CT, D), jnp.float32),
            pltpu.SemaphoreType.DMA,
            pltpu.SemaphoreType.DMA,
        ],
    )
    return f(ids2d, seg2d, token_table, posseg)


def kernel(input_ids, segment_ids, token_table, seg_table, pe):
    B_, L_ = input_ids.shape
    N = B_ * L_
    ids2d = input_ids.reshape(N // 128, 128).astype(jnp.int32)
    seg2d = segment_ids.reshape(N // 128, 128).astype(jnp.int32)
    posseg = (seg_table[:, None, :] + pe[0, :L_][None, :, :]).reshape(3 * L_, D)
    out = _sc_call(ids2d, seg2d, token_table, posseg)
    return out.reshape(B_, L_, D)


# speedup vs baseline: 2.3285x; 1.2316x over previous
import functools
import jax
import jax.numpy as jnp
from jax import lax
from jax.experimental import pallas as pl
from jax.experimental.pallas import tpu as pltpu
from jax.experimental.pallas import tpu_sc as plsc

D = 64
L = 200
NC = 2
NS = 16
NW = NC * NS

CT = 512
CG = CT // 128


def _body(ids_hbm, seg_hbm, tok_hbm, ps_hbm, out_hbm,
          idx_v, segv_v, idx2_v, tok_v, ps_v, sem0, sem1):
    wid = lax.axis_index("s") * NC + lax.axis_index("c")
    n_tok = ids_hbm.shape[0] * 128
    nt = n_tok // NW
    n_chunks = nt // CT
    w_row0 = wid * (nt // 128)

    iota16 = lax.iota(jnp.int32, 16)

    def chunk(c, carry):
        row0 = w_row0 + c * CG
        pltpu.sync_copy(ids_hbm.at[pl.ds(row0, CG)], idx_v)
        pltpu.sync_copy(seg_hbm.at[pl.ds(row0, CG)], segv_v)

        for g in range(CG):
            for u in range(8):
                fb = (row0 + g) * 128 + u * 16
                lv = lax.rem(iota16 + fb, L)
                sl = pl.ds(u * 16, 16)
                idx2_v[g, sl] = segv_v[g, sl] * L + lv

        cps = []
        for g in range(CG):
            cps.append(pltpu.async_copy(
                ps_hbm.at[idx2_v.at[g]], tok_v.at[pl.ds(g * 128, 128)], sem1))
        for cp in cps:
            cp.wait()
        cps = []
        for g in range(CG):
            cps.append(pltpu.async_copy(
                tok_hbm.at[idx_v.at[g]], tok_v.at[pl.ds(g * 128, 128)], sem0,
                add=True))
        for cp in cps:
            cp.wait()

        pltpu.sync_copy(tok_v, out_hbm.at[pl.ds(row0 * 128, CT)])
        return carry

    lax.fori_loop(0, n_chunks, chunk, 0)


@jax.jit
def _sc_call(ids2d, seg2d, token_table, posseg):
    n_tok = ids2d.shape[0] * 128
    mesh = plsc.VectorSubcoreMesh(core_axis_name="c", subcore_axis_name="s")
    f = pl.kernel(
        _body,
        out_type=jax.ShapeDtypeStruct((n_tok, D), jnp.float32),
        mesh=mesh,
        compiler_params=pltpu.CompilerParams(use_tc_tiling_on_sc=False),
        scratch_types=[
            pltpu.VMEM((CG, 128), jnp.int32),
            pltpu.VMEM((CG, 128), jnp.int32),
            pltpu.VMEM((CG, 128), jnp.int32),
            pltpu.VMEM((CT, D), jnp.float32),
            pltpu.VMEM((CT, D), jnp.float32),
            pltpu.SemaphoreType.DMA,
            pltpu.SemaphoreType.DMA,
        ],
    )
    return f(ids2d, seg2d, token_table, posseg)


def kernel(input_ids, segment_ids, token_table, seg_table, pe):
    B_, L_ = input_ids.shape
    N = B_ * L_
    ids2d = input_ids.reshape(N // 128, 128).astype(jnp.int32)
    seg2d = segment_ids.reshape(N // 128, 128).astype(jnp.int32)
    posseg = (seg_table[:, None, :] + pe[0, :L_][None, :, :]).reshape(3 * L_, D)
    out = _sc_call(ids2d, seg2d, token_table, posseg)
    return out.reshape(B_, L_, D)
